# baseline (speedup 1.0000x reference)
import jax
import jax.numpy as jnp
import math
from jax import lax
from jax.experimental import pallas as pl
from jax.experimental.pallas import tpu as pltpu

_VOCAB = 100000
_N = 1024
_SMOOTH = 0.1
_CONF = 1.0 - _SMOOTH
_S = _SMOOTH / (_VOCAB - 2)
_C0 = (_VOCAB - 2) * _S * math.log(_S) + _CONF * math.log(_CONF)

_CB = 4000


def _tc_fused_t(xt, t1):
    nsteps = _VOCAB // _CB

    def body(x_ref, t_ref, out_ref, acc_ref):
        k = pl.program_id(0)
        x = x_ref[...]
        t = t_ref[...]
        rowid = lax.broadcasted_iota(jnp.int32, (_CB, _N), 0) + k * _CB
        part = jnp.float32(-_S) * jnp.sum(x, axis=0, keepdims=True) + jnp.float32(
            _S - _CONF
        ) * jnp.sum(jnp.where(rowid == t, x, jnp.float32(0.0)), axis=0, keepdims=True)

        @pl.when(k == 0)
        def _():
            acc_ref[...] = part + jnp.float32(_S) * x[0:1, :]

        @pl.when(k > 0)
        def _():
            acc_ref[...] += part

        @pl.when(k == nsteps - 1)
        def _():
            maskf = (t != 0).astype(jnp.float32)
            out_ref[0, 0] = jnp.sum(maskf * acc_ref[...]) + jnp.float32(
                _C0
            ) * jnp.sum(maskf)

    return pl.pallas_call(
        body,
        grid=(nsteps,),
        in_specs=[
            pl.BlockSpec((_CB, _N), lambda k: (k, 0)),
            pl.BlockSpec((1, _N), lambda k: (0, 0)),
        ],
        out_specs=pl.BlockSpec(
            (1, 1), lambda k: (0, 0), memory_space=pltpu.SMEM
        ),
        out_shape=jax.ShapeDtypeStruct((1, 1), jnp.float32),
        scratch_shapes=[pltpu.VMEM((1, _N), jnp.float32)],
    )(xt, t1)


def kernel(pred_scores, target_ids):
    xt = pred_scores.T
    t1 = target_ids.astype(jnp.int32).reshape(1, _N)
    out = _tc_fused_t(xt, t1)
    return out[0, 0]
